# baseline (speedup 1.0000x reference)
import functools
import math

import jax
import jax.numpy as jnp
from jax import lax
from jax.experimental import pallas as pl
from jax.experimental.pallas import tpu as pltpu
from jax.experimental.pallas import tpu_sc as plsc

_NC, _NS = 2, 16
_NW = _NC * _NS
_CH_GRPS = 4
_B_GRPS = _NW // _CH_GRPS
_PITCH = 33


@functools.lru_cache(maxsize=None)
def _build(B, L, C, NP, NR):
    CPW = C // _CH_GRPS
    NPR = CPW // 2
    NB = B // _B_GRPS
    UNROLL = 8
    assert L % UNROLL == 0

    mesh = plsc.VectorSubcoreMesh(core_axis_name="c", subcore_axis_name="s")

    @functools.partial(
        pl.kernel,
        out_type=jax.ShapeDtypeStruct((B, C, L), jnp.float32),
        mesh=mesh,
        scratch_types=[
            pltpu.VMEM((NPR * NP,), jnp.int32),
            pltpu.VMEM((NPR * NR,), jnp.int32),
            pltpu.VMEM((L,), jnp.int32),
            pltpu.VMEM((L,), jnp.int32),
            pltpu.VMEM((L,), jnp.int32),
            pltpu.VMEM((L,), jnp.int32),
            pltpu.VMEM((L * _PITCH,), jnp.float32),
            pltpu.VMEM((CPW, L), jnp.float32),
            pltpu.VMEM((CPW, L), jnp.float32),
            pltpu.SemaphoreType.DMA,
            pltpu.SemaphoreType.DMA,
            pltpu.SemaphoreType.DMA,
            pltpu.SemaphoreType.DMA,
            pltpu.SemaphoreType.DMA,
            pltpu.SemaphoreType.DMA,
        ],
        compiler_params=pltpu.CompilerParams(
            needs_layout_passes=False, use_tc_tiling_on_sc=True),
    )
    def k(wpt_hbm, wrt_hbm, ph_hbm, pr_hbm, out_hbm,
          wp_v, wr_v, ip0, ir0, ip1, ir1, ot, o0, o1,
          sp0, sr0, sp1, sr1, so0, so1):
        wid = lax.axis_index("s") * _NC + lax.axis_index("c")
        g = wid % _CH_GRPS
        ch0 = g * CPW
        b0 = (wid // _CH_GRPS) * NB

        pltpu.sync_copy(wpt_hbm.at[pl.ds(g * NPR * NP, NPR * NP)], wp_v)
        pltpu.sync_copy(wrt_hbm.at[pl.ds(g * NPR * NR, NPR * NR)], wr_v)

        slots = ((ip0, ir0, o0, sp0, sr0, so0),
                 (ip1, ir1, o1, sp1, sr1, so1))

        for s in range(2):
            ip, ir, _, sp, sr, _ = slots[s]
            pltpu.async_copy(ph_hbm.at[pl.ds((b0 + s) * L, L)], ip, sp)
            pltpu.async_copy(pr_hbm.at[pl.ds((b0 + s) * L, L)], ir, sr)

        iota = jnp.arange(16, dtype=jnp.int32)
        iota_t = jnp.arange(16, dtype=jnp.int32) * _PITCH
        hi_mask = jnp.int32(-65536)

        def compute(ip, ir, o):
            def do_group(l0, n):
                tvp = ip[pl.ds(l0, 16)] * NPR
                tvr = ir[pl.ds(l0, 16)] * NPR
                st_base = iota + l0 * _PITCH
                dnums = lax.GatherDimensionNumbers(
                    offset_dims=(), collapsed_slice_dims=(0,),
                    start_index_map=(0,))
                def bcast_lane(tv, j):
                    sel = jnp.full((16, 1), j, jnp.int32)
                    return lax.gather(
                        tv, sel, dnums, (1,),
                        mode=lax.GatherScatterMode.PROMISE_IN_BOUNDS)
                for j in range(n):
                    bp = bcast_lane(tvp, j)
                    br = bcast_lane(tvr, j)
                    vp = plsc.load_gather(wp_v, [bp + iota])
                    vr = plsc.load_gather(wr_v, [br + iota])
                    hi = (plsc.bitcast(vp & hi_mask, jnp.float32)
                          + plsc.bitcast(vr & hi_mask, jnp.float32))
                    lo = (plsc.bitcast(vp << 16, jnp.float32)
                          + plsc.bitcast(vr << 16, jnp.float32))
                    hi_idx = st_base + j * _PITCH
                    plsc.store_scatter(ot, [hi_idx], hi)
                    plsc.store_scatter(ot, [hi_idx + 16], lo)

            def it_body(it, carry):
                do_group(it * 16, 16)
                return carry
            lax.fori_loop(0, (L // 16), it_body, 0)
            if L % 16:
                do_group(L - 16, 16)

            SPAN = 15 * _PITCH + 8
            for l0 in list(range(0, L - 16, 16)) + [L - 16]:
                for c in range(CPW):
                    base = l0 * _PITCH + c
                    base8 = base & ~7
                    v = plsc.load_gather(
                        ot.at[pl.ds(base8, SPAN)], [iota_t + (base - base8)])
                    o[c, pl.ds(l0, 16)] = v

        def it_body(it, carry):
            for s in range(2):
                ip, ir, o, sp, sr, so = slots[s]
                b = b0 + 2 * it + s
                pltpu.make_async_copy(
                    ph_hbm.at[pl.ds(b * L, L)], ip, sp).wait()
                pltpu.make_async_copy(
                    pr_hbm.at[pl.ds(b * L, L)], ir, sr).wait()

                @pl.when(it >= 1)
                def _wait_out():
                    pltpu.make_async_copy(
                        o, out_hbm.at[b - 2, pl.ds(ch0, CPW), :], so).wait()

                compute(ip, ir, o)
                pltpu.async_copy(
                    o, out_hbm.at[b, pl.ds(ch0, CPW), :], so)

                @pl.when(2 * it + s + 2 < NB)
                def _prefetch_idx():
                    pltpu.async_copy(
                        ph_hbm.at[pl.ds((b + 2) * L, L)], ip, sp)
                    pltpu.async_copy(
                        pr_hbm.at[pl.ds((b + 2) * L, L)], ir, sr)
            return carry

        lax.fori_loop(0, NB // 2, it_body, 0)

        for s in range(2):
            _, _, o, _, _, so = slots[s]
            b = b0 + NB - 2 + s
            pltpu.make_async_copy(
                o, out_hbm.at[b, pl.ds(ch0, CPW), :], so).wait()

    return k


def _pack_tokmajor(w):
    N, C = w.shape
    ws = w * jnp.float32(math.sqrt(C))
    bits = jax.lax.bitcast_convert_type(
        ws.astype(jnp.bfloat16), jnp.uint16).astype(jnp.uint32)
    b4 = bits.reshape(N, C // 32, 2, 16)
    word = (b4[:, :, 0, :] << 16) | b4[:, :, 1, :]
    packed = jnp.transpose(word, (1, 0, 2)).reshape(-1)
    return jax.lax.bitcast_convert_type(packed, jnp.int32)


def kernel(phoneme, prosody, phoneme_weight, prosody_weight):
    B, L = phoneme.shape
    NP, C = phoneme_weight.shape
    NR, _ = prosody_weight.shape
    k = _build(B, L, C, NP, NR)
    return k(_pack_tokmajor(phoneme_weight), _pack_tokmajor(prosody_weight),
             phoneme.reshape(-1), prosody.reshape(-1))


# speedup vs baseline: 2.7313x; 2.7313x over previous
import functools
import math

import jax
import jax.numpy as jnp
from jax import lax
from jax.experimental import pallas as pl
from jax.experimental.pallas import tpu as pltpu
from jax.experimental.pallas import tpu_sc as plsc

_NC, _NS = 2, 16
_NW = _NC * _NS
_CH_GRPS = 4
_B_GRPS = _NW // _CH_GRPS


@functools.lru_cache(maxsize=None)
def _build(B, L, C, NP, NR):
    CPW = C // _CH_GRPS
    NB = B // _B_GRPS
    FG = L // 16
    REM = L % 16

    mesh = plsc.VectorSubcoreMesh(core_axis_name="c", subcore_axis_name="s")

    @functools.partial(
        pl.kernel,
        out_type=jax.ShapeDtypeStruct((B, C, L), jnp.float32),
        mesh=mesh,
        scratch_types=[
            pltpu.VMEM((CPW // 2 * NP,), jnp.int32),
            pltpu.VMEM((CPW // 2 * NR,), jnp.int32),
            pltpu.VMEM((L,), jnp.int32),
            pltpu.VMEM((L,), jnp.int32),
            pltpu.VMEM((L,), jnp.int32),
            pltpu.VMEM((L,), jnp.int32),
            pltpu.VMEM((CPW, L), jnp.float32),
            pltpu.VMEM((CPW, L), jnp.float32),
            pltpu.SemaphoreType.DMA,
            pltpu.SemaphoreType.DMA,
            pltpu.SemaphoreType.DMA,
            pltpu.SemaphoreType.DMA,
            pltpu.SemaphoreType.DMA,
            pltpu.SemaphoreType.DMA,
        ],
        compiler_params=pltpu.CompilerParams(
            needs_layout_passes=False, use_tc_tiling_on_sc=True),
    )
    def k(wpt_hbm, wrt_hbm, ph_hbm, pr_hbm, out_hbm,
          wp_v, wr_v, ip0, ir0, ip1, ir1, o0, o1,
          sp0, sr0, sp1, sr1, so0, so1):
        wid = lax.axis_index("s") * _NC + lax.axis_index("c")
        ch0 = (wid % _CH_GRPS) * CPW
        b0 = (wid // _CH_GRPS) * NB

        cp0 = ch0 // 2
        pltpu.sync_copy(wpt_hbm.at[pl.ds(cp0 * NP, CPW // 2 * NP)], wp_v)
        pltpu.sync_copy(wrt_hbm.at[pl.ds(cp0 * NR, CPW // 2 * NR)], wr_v)

        slots = ((ip0, ir0, o0, sp0, sr0, so0),
                 (ip1, ir1, o1, sp1, sr1, so1))

        for s in range(2):
            ip, ir, _, sp, sr, _ = slots[s]
            pltpu.async_copy(ph_hbm.at[pl.ds((b0 + s) * L, L)], ip, sp)
            pltpu.async_copy(pr_hbm.at[pl.ds((b0 + s) * L, L)], ir, sr)

        def compute(ip, ir, o):
            NCP = CPW // 2
            hi_mask = jnp.int32(-65536)

            def do_group(l0):
                tp = ip[pl.ds(l0, 16)]
                tr = ir[pl.ds(l0, 16)]
                vps = [plsc.load_gather(wp_v.at[pl.ds(p * NP, NP)], [tp])
                       for p in range(NCP)]
                vrs = [plsc.load_gather(wr_v.at[pl.ds(p * NR, NR)], [tr])
                       for p in range(NCP)]
                for p in range(NCP):
                    hp = plsc.bitcast(vps[p] & hi_mask, jnp.float32)
                    hr = plsc.bitcast(vrs[p] & hi_mask, jnp.float32)
                    lp = plsc.bitcast(vps[p] << 16, jnp.float32)
                    lr = plsc.bitcast(vrs[p] << 16, jnp.float32)
                    o[2 * p, pl.ds(l0, 16)] = hp + hr
                    o[2 * p + 1, pl.ds(l0, 16)] = lp + lr

            def g_body(g, carry):
                do_group(g * 16)
                return carry
            lax.fori_loop(0, FG, g_body, 0)
            if REM:
                do_group(L - 16)

        def it_body(it, carry):
            for s in range(2):
                ip, ir, o, sp, sr, so = slots[s]
                b = b0 + 2 * it + s
                pltpu.make_async_copy(
                    ph_hbm.at[pl.ds(b * L, L)], ip, sp).wait()
                pltpu.make_async_copy(
                    pr_hbm.at[pl.ds(b * L, L)], ir, sr).wait()

                @pl.when(it >= 1)
                def _wait_out():
                    pltpu.make_async_copy(
                        o, out_hbm.at[b - 2, pl.ds(ch0, CPW), :], so).wait()

                compute(ip, ir, o)
                pltpu.async_copy(
                    o, out_hbm.at[b, pl.ds(ch0, CPW), :], so)

                @pl.when(2 * it + s + 2 < NB)
                def _prefetch_idx():
                    pltpu.async_copy(
                        ph_hbm.at[pl.ds((b + 2) * L, L)], ip, sp)
                    pltpu.async_copy(
                        pr_hbm.at[pl.ds((b + 2) * L, L)], ir, sr)
            return carry

        lax.fori_loop(0, NB // 2, it_body, 0)

        for s in range(2):
            _, _, o, _, _, so = slots[s]
            b = b0 + NB - 2 + s
            pltpu.make_async_copy(
                o, out_hbm.at[b, pl.ds(ch0, CPW), :], so).wait()

    return k


def _pack_pairs(w):
    C = w.shape[1]
    ws = jnp.transpose(w) * jnp.float32(math.sqrt(C))
    bits = jax.lax.bitcast_convert_type(
        ws.astype(jnp.bfloat16), jnp.uint16).astype(jnp.uint32)
    packed = (bits[0::2] << 16) | bits[1::2]
    return jax.lax.bitcast_convert_type(packed, jnp.int32).reshape(-1)


def kernel(phoneme, prosody, phoneme_weight, prosody_weight):
    B, L = phoneme.shape
    NP, C = phoneme_weight.shape
    NR, _ = prosody_weight.shape
    k = _build(B, L, C, NP, NR)
    return k(_pack_pairs(phoneme_weight), _pack_pairs(prosody_weight),
             phoneme.reshape(-1), prosody.reshape(-1))
